# baseline (speedup 1.0000x reference)
import functools

import jax
import jax.numpy as jnp
from jax import lax
from jax.experimental import pallas as pl
from jax.experimental.pallas import tpu as pltpu
from jax.experimental.pallas import tpu_sc as plsc

B = 16
G = 50
H = 52
W = 52
HW = H * W
A = 3
C = 80
IOU_THRESHOLD = 0.7
LAMBDA_OBJ = 5.0
LAMBDA_NOOBJ = 1.0
LAMBDA_COORD = 1.0

NW = 32
GPW = (B * G) // NW
NIDX = C * 32


def _cell_prep(cx, cy, w, h, awv):
    fx = jnp.clip(jnp.floor(cx * W), 0.0, W - 1)
    fy = jnp.clip(jnp.floor(cy * H), 0.0, H - 1)
    cxi = fx.astype(jnp.int32)
    cyi = fy.astype(jnp.int32)
    cellidx = cyi * W + cxi
    best = None
    bp = None
    for a in range(A):
        aww = awv[a : a + 1, 0:1]
        awh = awv[a : a + 1, 1:2]
        inter = jnp.minimum(w, aww) * jnp.minimum(h, awh)
        un = w * h + aww * awh - inter
        iou = inter / jnp.maximum(un, 1e-9)
        if a == 0:
            best = iou
            bp = jnp.zeros_like(cellidx)
        else:
            bp = jnp.where(iou > best, a, bp)
            best = jnp.maximum(iou, best)
    return fx, fy, cellidx, bp


def _loss_kernel(pr_ref, gtb_ref, gtbT_ref, aw_ref, out_ref):
    b = pl.program_id(0)
    awv = aw_ref[...]

    cx = gtb_ref[0, :, 0:1]
    cy = gtb_ref[0, :, 1:2]
    w = jnp.clip(gtb_ref[0, :, 2:3], 1e-4, None)
    h = jnp.clip(gtb_ref[0, :, 3:4], 1e-4, None)
    _, _, cellidx, bp = _cell_prep(cx, cy, w, h, awv)
    gx1 = cx - w * 0.5
    gx2 = cx + w * 0.5
    gy1 = cy - h * 0.5
    gy2 = cy + h * 0.5
    area_g = w * h

    cx_r = gtbT_ref[0, 0:1, :]
    cy_r = gtbT_ref[0, 1:2, :]
    w_r = jnp.clip(gtbT_ref[0, 2:3, :], 1e-4, None)
    h_r = jnp.clip(gtbT_ref[0, 3:4, :], 1e-4, None)
    fx_r, fy_r, cellidx_r, bp_r = _cell_prep(cx_r, cy_r, w_r, h_r, awv)
    dx_r = cx_r - fx_r * (1.0 / W)
    dy_r = cy_r - fy_r * (1.0 / H)
    awselw = jnp.where(bp_r == 0, awv[0:1, 0:1],
                       jnp.where(bp_r == 1, awv[1:2, 0:1], awv[2:3, 0:1]))
    awselh = jnp.where(bp_r == 0, awv[0:1, 1:2],
                       jnp.where(bp_r == 1, awv[1:2, 1:2], awv[2:3, 1:2]))
    gw_r = jnp.log(w_r) - jnp.log(awselw)
    gh_r = jnp.log(h_r) - jnp.log(awselh)

    iota_n = lax.broadcasted_iota(jnp.int32, (1, HW), 1)
    gxv = (iota_n % W).astype(jnp.float32) * (1.0 / W)
    gyv = (iota_n // W).astype(jnp.float32) * (1.0 / H)
    iota_n2 = lax.broadcasted_iota(jnp.int32, (HW, 1), 0)

    noobj = jnp.float32(0.0)
    gfeat = jnp.zeros((5, G), jnp.float32)

    for a in range(A):
        blk = pr_ref[0, pl.ds(5 * a, 5), :]
        xs = jax.nn.sigmoid(blk[0:1])
        ys = jax.nn.sigmoid(blk[1:2])
        tw = blk[2:3]
        th = blk[3:4]
        ob = jax.nn.sigmoid(blk[4:5])
        pwv = awv[a : a + 1, 0:1] * jnp.exp(tw)
        phv = awv[a : a + 1, 1:2] * jnp.exp(th)
        pcx = xs + gxv
        pcy = ys + gyv
        px1 = pcx - pwv * 0.5
        px2 = pcx + pwv * 0.5
        py1 = pcy - phv * 0.5
        py2 = pcy + phv * 0.5

        iw = jnp.clip(jnp.minimum(gx2, px2) - jnp.maximum(gx1, px1), 0.0, None)
        ih = jnp.clip(jnp.minimum(gy2, py2) - jnp.maximum(gy1, py1), 0.0, None)
        inter = iw * ih
        un = area_g + pwv * phv - inter
        pos = inter > IOU_THRESHOLD * jnp.maximum(un, 1e-9)
        posany = jnp.any(pos, axis=0, keepdims=True)

        Mb = (iota_n == cellidx) & (bp == a)
        assigned = jnp.any(Mb, axis=0, keepdims=True)
        negmask = ~(posany | assigned)
        noobj = noobj + jnp.sum(jnp.where(negmask, ob * ob, 0.0))

        Mt = ((iota_n2 == cellidx_r) & (bp_r == a)).astype(jnp.float32)
        feats = jnp.concatenate([xs, ys, tw, th, ob], axis=0)
        gfeat = gfeat + lax.dot_general(
            feats, Mt, (((1,), (0,)), ((), ())),
            preferred_element_type=jnp.float32,
            precision=lax.Precision.HIGHEST)

    sx = gfeat[0:1]
    sy = gfeat[1:2]
    gtw = gfeat[2:3]
    gth = gfeat[3:4]
    sob = gfeat[4:5]

    pcx_g = sx + fx_r * (1.0 / W)
    pcy_g = sy + fy_r * (1.0 / H)
    pw_g = awselw * jnp.exp(gtw)
    ph_g = awselh * jnp.exp(gth)
    iw_g = jnp.clip(jnp.minimum(cx_r + w_r * 0.5, pcx_g + pw_g * 0.5)
                    - jnp.maximum(cx_r - w_r * 0.5, pcx_g - pw_g * 0.5),
                    0.0, None)
    ih_g = jnp.clip(jnp.minimum(cy_r + h_r * 0.5, pcy_g + ph_g * 0.5)
                    - jnp.maximum(cy_r - h_r * 0.5, pcy_g - ph_g * 0.5),
                    0.0, None)
    inter_g = iw_g * ih_g
    un_g = w_r * h_r + pw_g * ph_g - inter_g
    tiou = inter_g / jnp.maximum(un_g, 1e-9)

    coord = jnp.sum((sx - dx_r) ** 2 + (sy - dy_r) ** 2
                    + (gtw - gw_r) ** 2 + (gth - gh_r) ** 2)
    obj = jnp.sum((sob - tiou) ** 2)

    total = LAMBDA_NOOBJ * noobj + LAMBDA_OBJ * obj + LAMBDA_COORD * coord
    prev = jnp.where(b == 0, jnp.zeros((1, 1), jnp.float32), out_ref[...])
    out_ref[...] = prev + total


def _box_losses_tc(p5, gtb, gtbT, aw2):
    out = pl.pallas_call(
        _loss_kernel,
        grid=(B,),
        in_specs=[
            pl.BlockSpec((1, A * 5, HW), lambda b: (b, 0, 0)),
            pl.BlockSpec((1, G, 4), lambda b: (b, 0, 0)),
            pl.BlockSpec((1, 4, G), lambda b: (b, 0, 0)),
            pl.BlockSpec((A, 2), lambda b: (0, 0)),
        ],
        out_specs=pl.BlockSpec((1, 1), lambda b: (0, 0)),
        out_shape=jax.ShapeDtypeStruct((1, 1), jnp.float32),
    )(p5, gtb, gtbT, aw2)
    return out[0, 0]


NCH = NIDX // 128


def _sc_cls_body(pred_hbm, gt_hbm, anch_hbm, out_hbm, *rest):
    gt_v = rest[0]
    anch_v = rest[1]
    idxbufs = rest[2 : 2 + NCH]
    valbufs = rest[2 + NCH : 2 + 2 * NCH]
    res_v = rest[2 + 2 * NCH]
    sem = rest[3 + 2 * NCH]
    wid = lax.axis_index("s") * 2 + lax.axis_index("c")

    pltpu.sync_copy(gt_hbm.at[wid], gt_v)
    pltpu.sync_copy(anch_hbm, anch_v)


    bases = []
    clss = []
    valids = []
    for k in range(2):
        o = k * 16
        cxv = gt_v[pl.ds(0 * 32 + o, 16)]
        cyv = gt_v[pl.ds(1 * 32 + o, 16)]
        wv = jnp.clip(gt_v[pl.ds(2 * 32 + o, 16)], 1e-4, None)
        hv = jnp.clip(gt_v[pl.ds(3 * 32 + o, 16)], 1e-4, None)
        clsv = gt_v[pl.ds(4 * 32 + o, 16)].astype(jnp.int32)
        validf = gt_v[pl.ds(5 * 32 + o, 16)]
        bv = gt_v[pl.ds(6 * 32 + o, 16)].astype(jnp.int32)
        cxi = jnp.clip((cxv * W).astype(jnp.int32), 0, W - 1)
        cyi = jnp.clip((cyv * H).astype(jnp.int32), 0, H - 1)
        best = None
        bp = None
        for a in range(A):
            aww = anch_v[pl.ds(a * 16, 16)]
            awh = anch_v[pl.ds(48 + a * 16, 16)]
            inter = jnp.minimum(wv, aww) * jnp.minimum(hv, awh)
            un = wv * hv + aww * awh - inter
            iou = inter / jnp.maximum(un, 1e-9)
            if a == 0:
                best = iou
                bp = jnp.zeros((16,), jnp.int32)
            else:
                bp = jnp.where(iou > best, a, bp)
                best = jnp.maximum(iou, best)
        base = (bv * (A * 85) + bp * 85 + 5) * HW + cyi * W + cxi
        base = jnp.where(validf > 0.5, base, 0)
        bases.append(base)
        clss.append(clsv)
        valids.append(validf)

    for p in range(2 * C):
        c, k = p // 2, p % 2
        j, off = p // 8, (p % 8) * 16
        idxbufs[j][pl.ds(off, 16)] = bases[k] + c * HW

    handles = [
        pltpu.async_copy(pred_hbm.at[idxbufs[j]], valbufs[j], sem)
        for j in range(NCH)
    ]
    for hd in handles:
        hd.wait()

    acc = jnp.zeros((16,), jnp.float32)
    for p in range(2 * C):
        c, k = p // 2, p % 2
        j, off = p // 8, (p % 8) * 16
        v = valbufs[j][pl.ds(off, 16)]
        s = 1.0 / (1.0 + jnp.exp(-v))
        oh = jnp.where(clss[k] == c, 1.0, 0.0)
        d = s - oh
        acc = acc + valids[k] * d * d

    res_v[...] = acc
    pltpu.sync_copy(res_v, out_hbm.at[wid])


@functools.lru_cache(maxsize=1)
def _make_cls_kernel():
    return pl.kernel(
        _sc_cls_body,
        out_type=jax.ShapeDtypeStruct((NW, 16), jnp.float32),
        mesh=plsc.VectorSubcoreMesh(core_axis_name="c", subcore_axis_name="s"),
        scratch_types=(
            [pltpu.VMEM((8 * 32,), jnp.float32),
             pltpu.VMEM((96,), jnp.float32)]
            + [pltpu.VMEM((128,), jnp.int32) for _ in range(NCH)]
            + [pltpu.VMEM((128,), jnp.float32) for _ in range(NCH)]
            + [pltpu.VMEM((16,), jnp.float32),
               pltpu.SemaphoreType.DMA]
        ),
    )


def _cls_loss_sc(pred_flat, gt6, anch16):
    return jnp.sum(_make_cls_kernel()(pred_flat, gt6, anch16))


@jax.jit
def kernel(pred, anchors, gt_boxes, gt_classes):
    p5 = pred.reshape(B, A, 85, HW)[:, :, :5].reshape(B, A * 5, HW)
    gtb = gt_boxes
    gtbT = gt_boxes.transpose(0, 2, 1)
    aw2 = anchors.reshape(A, 2)

    box_losses = _box_losses_tc(p5, gtb, gtbT, aw2)

    bimg = (jnp.arange(B * G, dtype=jnp.int32) // G).astype(jnp.float32)
    gt6 = jnp.concatenate(
        [gt_boxes.reshape(B * G, 4),
         gt_classes.reshape(B * G, 1).astype(jnp.float32),
         jnp.ones((B * G, 1), jnp.float32),
         bimg.reshape(B * G, 1),
         jnp.zeros((B * G, 1), jnp.float32)], axis=1)
    gt6 = gt6.reshape(NW, GPW, 8).transpose(0, 2, 1)
    gt6 = jnp.pad(gt6, ((0, 0), (0, 0), (0, 32 - GPW))).reshape(NW, 8 * 32)
    anch96 = jnp.broadcast_to(
        anchors.reshape(A, 2).T.reshape(A * 2, 1), (A * 2, 16)).reshape(96)
    pred_flat = pred.reshape(-1)
    cls_loss = _cls_loss_sc(pred_flat, gt6, anch96)

    return box_losses + cls_loss


# speedup vs baseline: 3.2278x; 3.2278x over previous
import functools

import jax
import jax.numpy as jnp
from jax import lax
from jax.experimental import pallas as pl
from jax.experimental.pallas import tpu as pltpu

B = 16
G = 50
H = 52
W = 52
HW = H * W
A = 3
C = 80
IOU_THRESHOLD = 0.7
LAMBDA_OBJ = 5.0
LAMBDA_NOOBJ = 1.0
LAMBDA_COORD = 1.0


def _cell_prep(cx, cy, w, h, awv):
    fx = jnp.clip(jnp.floor(cx * W), 0.0, W - 1)
    fy = jnp.clip(jnp.floor(cy * H), 0.0, H - 1)
    cxi = fx.astype(jnp.int32)
    cyi = fy.astype(jnp.int32)
    cellidx = cyi * W + cxi
    best = None
    bp = None
    for a in range(A):
        aww = awv[a : a + 1, 0:1]
        awh = awv[a : a + 1, 1:2]
        inter = jnp.minimum(w, aww) * jnp.minimum(h, awh)
        un = w * h + aww * awh - inter
        iou = inter / jnp.clip(un, 1e-9, None)
        if a == 0:
            best = iou
            bp = jnp.zeros_like(cellidx)
        else:
            bp = jnp.where(iou > best, a, bp)
            best = jnp.maximum(iou, best)
    return fx, fy, cxi, cyi, cellidx, bp


def _loss_kernel(pr_ref, gtb_ref, gtbT_ref, clsr_ref, aw_ref, out_ref):
    b = pl.program_id(0)
    awv = aw_ref[...]

    cx = gtb_ref[0, :, 0:1]
    cy = gtb_ref[0, :, 1:2]
    w = jnp.clip(gtb_ref[0, :, 2:3], 1e-4, None)
    h = jnp.clip(gtb_ref[0, :, 3:4], 1e-4, None)
    _, _, _, _, cellidx, bp = _cell_prep(cx, cy, w, h, awv)
    gx1 = cx - w * 0.5
    gx2 = cx + w * 0.5
    gy1 = cy - h * 0.5
    gy2 = cy + h * 0.5
    area_g = w * h

    cx_r = gtbT_ref[0, 0:1, :]
    cy_r = gtbT_ref[0, 1:2, :]
    w_r = jnp.clip(gtbT_ref[0, 2:3, :], 1e-4, None)
    h_r = jnp.clip(gtbT_ref[0, 3:4, :], 1e-4, None)
    fx_r, fy_r, _, _, cellidx_r, bp_r = _cell_prep(cx_r, cy_r, w_r, h_r, awv)
    dx_r = cx_r - fx_r * (1.0 / W)
    dy_r = cy_r - fy_r * (1.0 / H)
    awselw = jnp.where(bp_r == 0, awv[0:1, 0:1],
                       jnp.where(bp_r == 1, awv[1:2, 0:1], awv[2:3, 0:1]))
    awselh = jnp.where(bp_r == 0, awv[0:1, 1:2],
                       jnp.where(bp_r == 1, awv[1:2, 1:2], awv[2:3, 1:2]))
    gw_r = jnp.log(w_r) - jnp.log(awselw)
    gh_r = jnp.log(h_r) - jnp.log(awselh)
    clsid_r = clsr_ref[0].astype(jnp.int32)

    iota_n = lax.broadcasted_iota(jnp.int32, (1, HW), 1)
    gxv = (iota_n % W).astype(jnp.float32) * (1.0 / W)
    gyv = (iota_n // W).astype(jnp.float32) * (1.0 / H)
    iota_n2 = lax.broadcasted_iota(jnp.int32, (HW, 1), 0)
    iota_cls = lax.broadcasted_iota(jnp.int32, (C, 1), 0)

    noobj = jnp.float32(0.0)
    tgt_iou = jnp.zeros((G, 1), jnp.float32)
    obg = jnp.zeros((G, 1), jnp.float32)
    gfeat = jnp.zeros((4, G), jnp.float32)
    gcls = jnp.zeros((C, G), jnp.float32)

    for a in range(A):
        blk = pr_ref[0, pl.ds(85 * a, 5), :]
        xs = jax.nn.sigmoid(blk[0:1])
        ys = jax.nn.sigmoid(blk[1:2])
        tw = blk[2:3]
        th = blk[3:4]
        ob = jax.nn.sigmoid(blk[4:5])
        pwv = awv[a : a + 1, 0:1] * jnp.exp(tw)
        phv = awv[a : a + 1, 1:2] * jnp.exp(th)
        pcx = xs + gxv
        pcy = ys + gyv
        px1 = pcx - pwv * 0.5
        px2 = pcx + pwv * 0.5
        py1 = pcy - phv * 0.5
        py2 = pcy + phv * 0.5

        iw = jnp.clip(jnp.minimum(gx2, px2) - jnp.maximum(gx1, px1), 0.0, None)
        ih = jnp.clip(jnp.minimum(gy2, py2) - jnp.maximum(gy1, py1), 0.0, None)
        inter = iw * ih
        un = area_g + pwv * phv - inter
        iou = inter / jnp.clip(un, 1e-9, None)
        maxiou = jnp.max(iou, axis=0, keepdims=True)

        Mb = (iota_n == cellidx) & (bp == a)
        Mf = Mb.astype(jnp.float32)
        tgt_iou = tgt_iou + jnp.sum(iou * Mf, axis=1, keepdims=True)
        obg = obg + jnp.sum(ob * Mf, axis=1, keepdims=True)
        assigned = jnp.sum(Mf, axis=0, keepdims=True)
        negmask = (maxiou <= IOU_THRESHOLD) & (assigned <= 0.0)
        noobj = noobj + jnp.sum(jnp.where(negmask, ob * ob, 0.0))

        Mt = ((iota_n2 == cellidx_r) & (bp_r == a)).astype(jnp.float32)
        feats = jnp.concatenate([xs, ys, tw, th], axis=0)
        gfeat = gfeat + lax.dot_general(
            feats, Mt, (((1,), (0,)), ((), ())),
            preferred_element_type=jnp.float32,
            precision=lax.Precision.HIGHEST)
        clsblk = pr_ref[0, pl.ds(85 * a + 5, C), :]
        gcls = gcls + lax.dot_general(
            clsblk, Mt, (((1,), (0,)), ((), ())),
            preferred_element_type=jnp.float32,
            precision=lax.Precision.HIGHEST)

    sx = gfeat[0:1]
    sy = gfeat[1:2]
    gtw = gfeat[2:3]
    gth = gfeat[3:4]
    coord = jnp.sum((sx - dx_r) ** 2 + (sy - dy_r) ** 2
                    + (gtw - gw_r) ** 2 + (gth - gh_r) ** 2)
    obj = jnp.sum((obg - tgt_iou) ** 2)
    oh = (iota_cls == clsid_r).astype(jnp.float32)
    scls = jax.nn.sigmoid(gcls)
    cls_loss = jnp.sum((scls - oh) ** 2)

    total = (cls_loss + LAMBDA_NOOBJ * noobj + LAMBDA_OBJ * obj
             + LAMBDA_COORD * coord)
    row0 = jax.lax.broadcasted_iota(jnp.int32, (8, 1), 0) == 0
    out_ref[...] = jnp.where(row0, total, 0.0).astype(jnp.float32)


@jax.jit
def kernel(pred, anchors, gt_boxes, gt_classes):
    pr = pred.reshape(B, A * 85, HW)
    gtb = gt_boxes
    gtbT = gt_boxes.transpose(0, 2, 1)
    clsr = gt_classes.astype(jnp.float32).reshape(B, 1, G)
    aw2 = anchors.reshape(A, 2)

    out = pl.pallas_call(
        _loss_kernel,
        grid=(B,),
        in_specs=[
            pl.BlockSpec((1, A * 85, HW), lambda b: (b, 0, 0)),
            pl.BlockSpec((1, G, 4), lambda b: (b, 0, 0)),
            pl.BlockSpec((1, 4, G), lambda b: (b, 0, 0)),
            pl.BlockSpec((1, 1, G), lambda b: (b, 0, 0)),
            pl.BlockSpec((A, 2), lambda b: (0, 0)),
        ],
        out_specs=pl.BlockSpec((8, 1), lambda b: (b, 0)),
        out_shape=jax.ShapeDtypeStruct((B * 8, 1), jnp.float32),
        compiler_params=pltpu.CompilerParams(
            dimension_semantics=("parallel",)),
    )(pr, gtb, gtbT, clsr, aw2)
    return jnp.sum(out)
